# baseline (speedup 1.0000x reference)
import functools

import jax
import jax.numpy as jnp
from jax import lax
from jax.experimental import pallas as pl
from jax.experimental.pallas import tpu as pltpu
from jax.experimental.pallas import tpu_sc as plsc

NUM_EMBEDDINGS = 1024
EMBEDDING_DIM = 64
COMMITMENT_COST = 0.25
CONTRIB_RATE = 0.05

ROWS = 9216
BLK = 3072
GRID = ROWS // BLK
_LOSS_SCALE = (1.0 + COMMITMENT_COST) / float(ROWS * EMBEDDING_DIM)

_SC_INFO = plsc.get_sparse_core_info()
_NC = _SC_INFO.num_cores
_NS = _SC_INFO.num_subcores
_NW = _NC * _NS
_BPW = ROWS // _NW


def _vq_tc_kernel(x_ref, w_ref, idx_ref, loss_ref, wpad_ref,
                  wsq_ref, w2_ref, ids_ref):
    i = pl.program_id(0)
    x = x_ref[...]
    w = w_ref[...]

    @pl.when(i == 0)
    def _():
        wsq_ref[...] = jnp.sum(w * w, axis=1)[None, :]
        w2_ref[...] = w * -2.0
        ids_ref[...] = jax.lax.broadcasted_iota(
            jnp.int32, (1, NUM_EMBEDDINGS), 1).astype(jnp.float32)
        wpad_ref[...] = jnp.concatenate((w, w), axis=1)

    xsq = jnp.sum(x * x, axis=1, keepdims=True)
    xw2 = jax.lax.dot_general(
        x, w2_ref[...], (((1,), (1,)), ((), ())),
        preferred_element_type=jnp.float32)
    dist = (xsq + wsq_ref[...]) + xw2

    dmin = jnp.min(dist, axis=1, keepdims=True)
    m = dist == dmin
    idxf = jnp.min(jnp.where(m, ids_ref[...], float(NUM_EMBEDDINGS)),
                   axis=1, keepdims=True)
    idx_ref[...] = idxf.astype(jnp.int32)

    part = jnp.sum(dmin, axis=(0, 1), keepdims=True)
    prev = jnp.where(i == 0, 0.0, loss_ref[...])
    acc = prev + part
    loss_ref[...] = jnp.where(i == GRID - 1, acc * _LOSS_SCALE, acc)


def _sc_gather_blend(x_hbm, w_hbm, idx_hbm, out_hbm,
                     idx_v, x_v, rows_v, out_v, sem):
    wid = lax.axis_index("s") * _NC + lax.axis_index("c")
    base = wid * _BPW
    pltpu.sync_copy(idx_hbm.at[pl.ds(base, _BPW)], idx_v)
    pltpu.sync_copy(x_hbm.at[pl.ds(base, _BPW), :], x_v)
    pltpu.async_copy(w_hbm.at[idx_v], rows_v, sem).wait()

    def body(r, _):
        for c in range(EMBEDDING_DIM // 16):
            sl = pl.ds(c * 16, 16)
            xv = x_v[r, sl]
            qv = rows_v[r, sl]
            out_v[r, sl] = (xv * (1.0 - CONTRIB_RATE)
                            + (qv - xv) * CONTRIB_RATE)
        return 0

    lax.fori_loop(0, _BPW, body, 0)
    pltpu.sync_copy(out_v, out_hbm.at[pl.ds(base, _BPW), :])


@functools.partial(jax.jit, static_argnames=())
def kernel(inputs, W):
    input_shape = inputs.shape
    flat = inputs.reshape(ROWS, EMBEDDING_DIM)
    idx, loss, wpad = pl.pallas_call(
        _vq_tc_kernel,
        grid=(GRID,),
        in_specs=[
            pl.BlockSpec((BLK, EMBEDDING_DIM), lambda i: (i, 0)),
            pl.BlockSpec((NUM_EMBEDDINGS, EMBEDDING_DIM), lambda i: (0, 0)),
        ],
        out_specs=[
            pl.BlockSpec((BLK, 1), lambda i: (i, 0)),
            pl.BlockSpec((1, 1), lambda i: (0, 0)),
            pl.BlockSpec((NUM_EMBEDDINGS, 128), lambda i: (0, 0)),
        ],
        out_shape=[
            jax.ShapeDtypeStruct((ROWS, 1), jnp.int32),
            jax.ShapeDtypeStruct((1, 1), jnp.float32),
            jax.ShapeDtypeStruct((NUM_EMBEDDINGS, 128), jnp.float32),
        ],
        scratch_shapes=[
            pltpu.VMEM((1, NUM_EMBEDDINGS), jnp.float32),
            pltpu.VMEM((NUM_EMBEDDINGS, EMBEDDING_DIM), jnp.float32),
            pltpu.VMEM((1, NUM_EMBEDDINGS), jnp.float32),
        ],
        compiler_params=pltpu.CompilerParams(
            dimension_semantics=("arbitrary",)),
    )(flat, W)

    sc = pl.kernel(
        _sc_gather_blend,
        out_type=jax.ShapeDtypeStruct((ROWS, EMBEDDING_DIM), jnp.float32),
        mesh=plsc.VectorSubcoreMesh(core_axis_name="c", subcore_axis_name="s"),
        scratch_types=[
            pltpu.VMEM((_BPW,), jnp.int32),
            pltpu.VMEM((_BPW, EMBEDDING_DIM), jnp.float32),
            pltpu.VMEM((_BPW, 128), jnp.float32),
            pltpu.VMEM((_BPW, EMBEDDING_DIM), jnp.float32),
            pltpu.SemaphoreType.DMA,
        ],
    )
    out = sc(flat, wpad, idx.reshape(ROWS))
    return out.reshape(input_shape), idx, loss[0, 0]


# speedup vs baseline: 1.5344x; 1.5344x over previous
import functools

import jax
import jax.numpy as jnp
from jax.experimental import pallas as pl
from jax.experimental.pallas import tpu as pltpu

NUM_EMBEDDINGS = 1024
EMBEDDING_DIM = 64
COMMITMENT_COST = 0.25
CONTRIB_RATE = 0.05

ROWS = 9216
BLK = 3072
GRID = ROWS // BLK
_LOSS_SCALE = (1.0 + COMMITMENT_COST) / float(ROWS * EMBEDDING_DIM)


def _vq_kernel(x_ref, w_ref, out_ref, idx_ref, loss_ref,
               wsq_ref, w2_ref, ids_ref):
    i = pl.program_id(0)
    x = x_ref[...]
    w = w_ref[...]

    @pl.when(i == 0)
    def _():
        wsq_ref[...] = jnp.sum(w * w, axis=1)[None, :]
        w2_ref[...] = w * -2.0
        ids_ref[...] = jax.lax.broadcasted_iota(
            jnp.int32, (1, NUM_EMBEDDINGS), 1).astype(jnp.float32)

    HB = BLK // 2
    parts = []
    for h in range(2):
        rs = pl.ds(h * HB, HB)
        xh = x[h * HB:(h + 1) * HB, :]

        xsq = jnp.sum(xh * xh, axis=1, keepdims=True)
        xw2 = jax.lax.dot_general(
            xh, w2_ref[...], (((1,), (1,)), ((), ())),
            preferred_element_type=jnp.float32)
        dist = (xsq + wsq_ref[...]) + xw2

        dmin = jnp.min(dist, axis=1, keepdims=True)
        m = dist == dmin
        idxf = jnp.min(jnp.where(m, ids_ref[...], float(NUM_EMBEDDINGS)),
                       axis=1, keepdims=True)
        idx_ref[rs, :] = idxf.astype(jnp.int32)

        enc = jnp.where(m, 1.0, 0.0)
        quant = jax.lax.dot_general(
            enc, w, (((1,), (0,)), ((), ())),
            preferred_element_type=jnp.float32)
        out_ref[rs, :] = (xh * (1.0 - CONTRIB_RATE)
                          + (quant - xh) * CONTRIB_RATE)

        parts.append(jnp.sum(dmin, axis=(0, 1), keepdims=True))

    part = parts[0] + parts[1]
    prev = jnp.where(i == 0, 0.0, loss_ref[...])
    acc = prev + part
    loss_ref[...] = jnp.where(i == GRID - 1, acc * _LOSS_SCALE, acc)


@functools.partial(jax.jit, static_argnames=())
def kernel(inputs, W):
    input_shape = inputs.shape
    flat = inputs.reshape(ROWS, EMBEDDING_DIM)
    out, idx, loss = pl.pallas_call(
        _vq_kernel,
        grid=(GRID,),
        in_specs=[
            pl.BlockSpec((BLK, EMBEDDING_DIM), lambda i: (i, 0)),
            pl.BlockSpec((NUM_EMBEDDINGS, EMBEDDING_DIM), lambda i: (0, 0)),
        ],
        out_specs=[
            pl.BlockSpec((BLK, EMBEDDING_DIM), lambda i: (i, 0)),
            pl.BlockSpec((BLK, 1), lambda i: (i, 0)),
            pl.BlockSpec((1, 1), lambda i: (0, 0)),
        ],
        out_shape=[
            jax.ShapeDtypeStruct((ROWS, EMBEDDING_DIM), jnp.float32),
            jax.ShapeDtypeStruct((ROWS, 1), jnp.int32),
            jax.ShapeDtypeStruct((1, 1), jnp.float32),
        ],
        scratch_shapes=[
            pltpu.VMEM((1, NUM_EMBEDDINGS), jnp.float32),
            pltpu.VMEM((NUM_EMBEDDINGS, EMBEDDING_DIM), jnp.float32),
            pltpu.VMEM((1, NUM_EMBEDDINGS), jnp.float32),
        ],
        compiler_params=pltpu.CompilerParams(
            dimension_semantics=("arbitrary",)),
    )(flat, W)
    return out.reshape(input_shape), idx, loss[0, 0]


# speedup vs baseline: 1.6072x; 1.0475x over previous
import functools

import jax
import jax.numpy as jnp
from jax.experimental import pallas as pl
from jax.experimental.pallas import tpu as pltpu

NUM_EMBEDDINGS = 1024
EMBEDDING_DIM = 64
COMMITMENT_COST = 0.25
CONTRIB_RATE = 0.05

ROWS = 9216
BLK = 3072
GRID = ROWS // BLK
_LOSS_SCALE = (1.0 + COMMITMENT_COST) / float(ROWS * EMBEDDING_DIM)


def _vq_kernel(x_ref, w_ref, out_ref, idx_ref, loss_ref,
               wsq_ref, w2_ref, ids_ref):
    i = pl.program_id(0)
    x = x_ref[...]
    w = w_ref[...]

    @pl.when(i == 0)
    def _():
        wsq_ref[...] = jnp.sum(w * w, axis=1)[None, :]
        w2_ref[...] = w * -2.0
        ids_ref[...] = jax.lax.broadcasted_iota(
            jnp.int32, (1, NUM_EMBEDDINGS), 1).astype(jnp.float32)

    HB = BLK // 4
    parts = []
    for h in range(4):
        rs = pl.ds(h * HB, HB)
        xh = x[h * HB:(h + 1) * HB, :]

        xsq = jnp.sum(xh * xh, axis=1, keepdims=True)
        xw2 = jax.lax.dot_general(
            xh, w2_ref[...], (((1,), (1,)), ((), ())),
            preferred_element_type=jnp.float32)
        dist = (xsq + wsq_ref[...]) + xw2

        dmin = jnp.min(dist, axis=1, keepdims=True)
        m = dist == dmin
        idxf = jnp.min(jnp.where(m, ids_ref[...], float(NUM_EMBEDDINGS)),
                       axis=1, keepdims=True)
        idx_ref[rs, :] = idxf.astype(jnp.int32)

        enc = jnp.where(m, 1.0, 0.0)
        quant = jax.lax.dot_general(
            enc, w, (((1,), (0,)), ((), ())),
            preferred_element_type=jnp.float32)
        out_ref[rs, :] = (xh * (1.0 - CONTRIB_RATE)
                          + (quant - xh) * CONTRIB_RATE)

        parts.append(jnp.sum(dmin, axis=(0, 1), keepdims=True))

    part = (parts[0] + parts[1]) + (parts[2] + parts[3])
    prev = jnp.where(i == 0, 0.0, loss_ref[...])
    acc = prev + part
    loss_ref[...] = jnp.where(i == GRID - 1, acc * _LOSS_SCALE, acc)


@functools.partial(jax.jit, static_argnames=())
def kernel(inputs, W):
    input_shape = inputs.shape
    flat = inputs.reshape(ROWS, EMBEDDING_DIM)
    out, idx, loss = pl.pallas_call(
        _vq_kernel,
        grid=(GRID,),
        in_specs=[
            pl.BlockSpec((BLK, EMBEDDING_DIM), lambda i: (i, 0)),
            pl.BlockSpec((NUM_EMBEDDINGS, EMBEDDING_DIM), lambda i: (0, 0)),
        ],
        out_specs=[
            pl.BlockSpec((BLK, EMBEDDING_DIM), lambda i: (i, 0)),
            pl.BlockSpec((BLK, 1), lambda i: (i, 0)),
            pl.BlockSpec((1, 1), lambda i: (0, 0)),
        ],
        out_shape=[
            jax.ShapeDtypeStruct((ROWS, EMBEDDING_DIM), jnp.float32),
            jax.ShapeDtypeStruct((ROWS, 1), jnp.int32),
            jax.ShapeDtypeStruct((1, 1), jnp.float32),
        ],
        scratch_shapes=[
            pltpu.VMEM((1, NUM_EMBEDDINGS), jnp.float32),
            pltpu.VMEM((NUM_EMBEDDINGS, EMBEDDING_DIM), jnp.float32),
            pltpu.VMEM((1, NUM_EMBEDDINGS), jnp.float32),
        ],
        compiler_params=pltpu.CompilerParams(
            dimension_semantics=("arbitrary",)),
    )(flat, W)
    return out.reshape(input_shape), idx, loss[0, 0]


# speedup vs baseline: 1.6504x; 1.0269x over previous
import functools

import jax
import jax.numpy as jnp
from jax.experimental import pallas as pl
from jax.experimental.pallas import tpu as pltpu

NUM_EMBEDDINGS = 1024
EMBEDDING_DIM = 64
COMMITMENT_COST = 0.25
CONTRIB_RATE = 0.05

ROWS = 9216
BLK = 3072
GRID = ROWS // BLK
_LOSS_SCALE = (1.0 + COMMITMENT_COST) / float(ROWS * EMBEDDING_DIM)


def _vq_kernel(x_ref, w_ref, out_ref, idx_ref, loss_ref,
               wsq_ref, w2_ref, ids_ref):
    i = pl.program_id(0)
    x = x_ref[...]
    w = w_ref[...]

    @pl.when(i == 0)
    def _():
        wsq_ref[...] = jnp.sum(w * w, axis=1)[None, :]
        w2_ref[...] = w * -2.0
        ids_ref[...] = jax.lax.broadcasted_iota(
            jnp.int32, (1, NUM_EMBEDDINGS), 1).astype(jnp.float32)

    HB = BLK // 8
    parts = []
    for h in range(8):
        rs = pl.ds(h * HB, HB)
        xh = x[h * HB:(h + 1) * HB, :]

        xsq = jnp.sum(xh * xh, axis=1, keepdims=True)
        xw2 = jax.lax.dot_general(
            xh, w2_ref[...], (((1,), (1,)), ((), ())),
            preferred_element_type=jnp.float32)
        dist = (xsq + wsq_ref[...]) + xw2

        dmin = jnp.min(dist, axis=1, keepdims=True)
        m = dist == dmin
        idxf = jnp.min(jnp.where(m, ids_ref[...], float(NUM_EMBEDDINGS)),
                       axis=1, keepdims=True)
        idx_ref[rs, :] = idxf.astype(jnp.int32)

        enc = jnp.where(m, 1.0, 0.0)
        quant = jax.lax.dot_general(
            enc, w, (((1,), (0,)), ((), ())),
            preferred_element_type=jnp.float32)
        out_ref[rs, :] = (xh * (1.0 - CONTRIB_RATE)
                          + (quant - xh) * CONTRIB_RATE)

        parts.append(jnp.sum(dmin, axis=(0, 1), keepdims=True))

    part = ((parts[0] + parts[1]) + (parts[2] + parts[3])) + (
        (parts[4] + parts[5]) + (parts[6] + parts[7]))
    prev = jnp.where(i == 0, 0.0, loss_ref[...])
    acc = prev + part
    loss_ref[...] = jnp.where(i == GRID - 1, acc * _LOSS_SCALE, acc)


@functools.partial(jax.jit, static_argnames=())
def kernel(inputs, W):
    input_shape = inputs.shape
    flat = inputs.reshape(ROWS, EMBEDDING_DIM)
    out, idx, loss = pl.pallas_call(
        _vq_kernel,
        grid=(GRID,),
        in_specs=[
            pl.BlockSpec((BLK, EMBEDDING_DIM), lambda i: (i, 0)),
            pl.BlockSpec((NUM_EMBEDDINGS, EMBEDDING_DIM), lambda i: (0, 0)),
        ],
        out_specs=[
            pl.BlockSpec((BLK, EMBEDDING_DIM), lambda i: (i, 0)),
            pl.BlockSpec((BLK, 1), lambda i: (i, 0)),
            pl.BlockSpec((1, 1), lambda i: (0, 0)),
        ],
        out_shape=[
            jax.ShapeDtypeStruct((ROWS, EMBEDDING_DIM), jnp.float32),
            jax.ShapeDtypeStruct((ROWS, 1), jnp.int32),
            jax.ShapeDtypeStruct((1, 1), jnp.float32),
        ],
        scratch_shapes=[
            pltpu.VMEM((1, NUM_EMBEDDINGS), jnp.float32),
            pltpu.VMEM((NUM_EMBEDDINGS, EMBEDDING_DIM), jnp.float32),
            pltpu.VMEM((1, NUM_EMBEDDINGS), jnp.float32),
        ],
        compiler_params=pltpu.CompilerParams(
            dimension_semantics=("arbitrary",)),
    )(flat, W)
    return out.reshape(input_shape), idx, loss[0, 0]


# speedup vs baseline: 1.7053x; 1.0332x over previous
import functools

import jax
import jax.numpy as jnp
from jax.experimental import pallas as pl
from jax.experimental.pallas import tpu as pltpu

NUM_EMBEDDINGS = 1024
EMBEDDING_DIM = 64
COMMITMENT_COST = 0.25
CONTRIB_RATE = 0.05

ROWS = 9216
BLK = 3072
GRID = ROWS // BLK
_LOSS_SCALE = (1.0 + COMMITMENT_COST) / float(ROWS * EMBEDDING_DIM)


def _vq_kernel(x_ref, w_ref, out_ref, idx_ref, loss_ref,
               wsq_ref, w2_ref, ids_ref):
    i = pl.program_id(0)
    x = x_ref[...]
    w = w_ref[...]

    @pl.when(i == 0)
    def _():
        wsq_ref[...] = jnp.sum(w * w, axis=1)[None, :]
        w2_ref[...] = w * -2.0
        ids_ref[...] = jax.lax.broadcasted_iota(
            jnp.int32, (1, NUM_EMBEDDINGS), 1).astype(jnp.float32)

    HB = BLK // 16
    parts = []
    for h in range(16):
        rs = pl.ds(h * HB, HB)
        xh = x[h * HB:(h + 1) * HB, :]

        xsq = jnp.sum(xh * xh, axis=1, keepdims=True)
        xw2 = jax.lax.dot_general(
            xh, w2_ref[...], (((1,), (1,)), ((), ())),
            preferred_element_type=jnp.float32)
        dist = (xsq + wsq_ref[...]) + xw2

        dmin = jnp.min(dist, axis=1, keepdims=True)
        m = dist == dmin
        idxf = jnp.min(jnp.where(m, ids_ref[...], float(NUM_EMBEDDINGS)),
                       axis=1, keepdims=True)
        idx_ref[rs, :] = idxf.astype(jnp.int32)

        enc = jnp.where(m, 1.0, 0.0)
        quant = jax.lax.dot_general(
            enc, w, (((1,), (0,)), ((), ())),
            preferred_element_type=jnp.float32)
        out_ref[rs, :] = (xh * (1.0 - CONTRIB_RATE)
                          + (quant - xh) * CONTRIB_RATE)

        parts.append(jnp.sum(dmin, axis=(0, 1), keepdims=True))

    while len(parts) > 1:
        parts = [parts[k] + parts[k + 1] for k in range(0, len(parts), 2)]
    part = parts[0]
    prev = jnp.where(i == 0, 0.0, loss_ref[...])
    acc = prev + part
    loss_ref[...] = jnp.where(i == GRID - 1, acc * _LOSS_SCALE, acc)


@functools.partial(jax.jit, static_argnames=())
def kernel(inputs, W):
    input_shape = inputs.shape
    flat = inputs.reshape(ROWS, EMBEDDING_DIM)
    out, idx, loss = pl.pallas_call(
        _vq_kernel,
        grid=(GRID,),
        in_specs=[
            pl.BlockSpec((BLK, EMBEDDING_DIM), lambda i: (i, 0)),
            pl.BlockSpec((NUM_EMBEDDINGS, EMBEDDING_DIM), lambda i: (0, 0)),
        ],
        out_specs=[
            pl.BlockSpec((BLK, EMBEDDING_DIM), lambda i: (i, 0)),
            pl.BlockSpec((BLK, 1), lambda i: (i, 0)),
            pl.BlockSpec((1, 1), lambda i: (0, 0)),
        ],
        out_shape=[
            jax.ShapeDtypeStruct((ROWS, EMBEDDING_DIM), jnp.float32),
            jax.ShapeDtypeStruct((ROWS, 1), jnp.int32),
            jax.ShapeDtypeStruct((1, 1), jnp.float32),
        ],
        scratch_shapes=[
            pltpu.VMEM((1, NUM_EMBEDDINGS), jnp.float32),
            pltpu.VMEM((NUM_EMBEDDINGS, EMBEDDING_DIM), jnp.float32),
            pltpu.VMEM((1, NUM_EMBEDDINGS), jnp.float32),
        ],
        compiler_params=pltpu.CompilerParams(
            dimension_semantics=("arbitrary",)),
    )(flat, W)
    return out.reshape(input_shape), idx, loss[0, 0]
